# initial kernel scaffold (unmeasured)
import jax
import jax.numpy as jnp
from jax import lax
from jax.experimental import pallas as pl
from jax.experimental.pallas import tpu as pltpu

N_DEV = 4
B = 2
SQ_SHARD = 256
SQ = SQ_SHARD * N_DEV
D_MODEL = 768
H_LOC = 4
DH = 64
HD_LOC = H_LOC * DH


def _body(x_ref, wq_ref, wk_ref, wv_ref, wo_ref, out_ref,
          xg_ref, qp_ref, kp_ref, vp_ref, ctx_ref, pout_ref, po_ref, rs_ref,
          ag_send, ag_recv, rs_send, rs_recv):
    my = lax.axis_index("i")
    left = lax.rem(my + N_DEV - 1, N_DEV)
    right = lax.rem(my + 1, N_DEV)

    barrier = pltpu.get_barrier_semaphore()
    pl.semaphore_signal(barrier, inc=1, device_id=(left,),
                        device_id_type=pl.DeviceIdType.MESH)
    pl.semaphore_signal(barrier, inc=1, device_id=(right,),
                        device_id_type=pl.DeviceIdType.MESH)
    pl.semaphore_wait(barrier, 2)

    xg_ref[0] = x_ref[...]
    for h in range(N_DEV - 1):
        rdma = pltpu.make_async_remote_copy(
            src_ref=xg_ref.at[h],
            dst_ref=xg_ref.at[h + 1],
            send_sem=ag_send.at[h],
            recv_sem=ag_recv.at[h],
            device_id=(right,),
            device_id_type=pl.DeviceIdType.MESH,
        )
        rdma.start()
        rdma.wait()

    for r in range(N_DEV):
        blk = lax.rem(my + (N_DEV - r), N_DEV)
        row0 = blk * SQ_SHARD
        for b in range(B):
            xb = xg_ref[r, b]
            qp_ref[b, pl.ds(row0, SQ_SHARD), :] = jnp.dot(
                xb, wq_ref[...], preferred_element_type=jnp.float32)
            kp_ref[b, pl.ds(row0, SQ_SHARD), :] = jnp.dot(
                xb, wk_ref[...], preferred_element_type=jnp.float32)
            vp_ref[b, pl.ds(row0, SQ_SHARD), :] = jnp.dot(
                xb, wv_ref[...], preferred_element_type=jnp.float32)

    rowp = lax.broadcasted_iota(jnp.float32, (SQ, DH), 0)
    lane = lax.broadcasted_iota(jnp.int32, (SQ, DH), 1)
    k2 = (lane // 2).astype(jnp.float32)
    inv = jnp.exp(k2 * (-2.0 / DH) * jnp.log(10000.0))
    theta = rowp * inv
    cosv = jnp.cos(theta)
    sinv = jnp.sin(theta)
    ar = lax.broadcasted_iota(jnp.int32, (DH, DH), 0)
    ac = lax.broadcasted_iota(jnp.int32, (DH, DH), 1)
    rmat = jnp.where(
        (ar == ac + 1) & (ar % 2 == 1), -1.0,
        jnp.where((ac == ar + 1) & (ar % 2 == 0), 1.0, 0.0),
    ).astype(jnp.float32)

    for b in range(B):
        for h in range(H_LOC):
            q = qp_ref[b, :, pl.ds(h * DH, DH)]
            k = kp_ref[b, :, pl.ds(h * DH, DH)]
            v = vp_ref[b, :, pl.ds(h * DH, DH)]
            qr = q * cosv + jnp.dot(
                q, rmat, preferred_element_type=jnp.float32) * sinv
            kr = k * cosv + jnp.dot(
                k, rmat, preferred_element_type=jnp.float32) * sinv
            s = lax.dot_general(
                qr, kr, (((1,), (1,)), ((), ())),
                preferred_element_type=jnp.float32) * 0.125
            m = jnp.max(s, axis=1, keepdims=True)
            w = jnp.exp(s - m)
            w = w / jnp.sum(w, axis=1, keepdims=True)
            ctx_ref[:, pl.ds(h * DH, DH)] = jnp.dot(
                w, v, preferred_element_type=jnp.float32)
        pout_ref[...] = jnp.dot(
            ctx_ref[...], wo_ref[...], preferred_element_type=jnp.float32)
        for r in range(N_DEV):
            blk = lax.rem(my + (N_DEV - r), N_DEV)
            po_ref[r, b] = pout_ref[pl.ds(blk * SQ_SHARD, SQ_SHARD), :]

    for s in range(N_DEV - 1):
        rdma = pltpu.make_async_remote_copy(
            src_ref=po_ref.at[s + 1],
            dst_ref=rs_ref.at[s],
            send_sem=rs_send.at[s],
            recv_sem=rs_recv.at[s],
            device_id=(right,),
            device_id_type=pl.DeviceIdType.MESH,
        )
        rdma.start()
        rdma.wait()
        dst = (s + 2) % N_DEV
        po_ref[dst] = po_ref[dst] + rs_ref[s]

    out_ref[...] = po_ref[0]


def kernel(x, Wq, Wk, Wv, Wo):
    return pl.pallas_call(
        _body,
        out_shape=jax.ShapeDtypeStruct((B, SQ_SHARD, D_MODEL), jnp.float32),
        in_specs=[pl.BlockSpec(memory_space=pltpu.VMEM)] * 5,
        out_specs=pl.BlockSpec(memory_space=pltpu.VMEM),
        scratch_shapes=[
            pltpu.VMEM((N_DEV, B, SQ_SHARD, D_MODEL), jnp.float32),
            pltpu.VMEM((B, SQ, HD_LOC), jnp.float32),
            pltpu.VMEM((B, SQ, HD_LOC), jnp.float32),
            pltpu.VMEM((B, SQ, HD_LOC), jnp.float32),
            pltpu.VMEM((SQ, HD_LOC), jnp.float32),
            pltpu.VMEM((SQ, D_MODEL), jnp.float32),
            pltpu.VMEM((N_DEV, B, SQ_SHARD, D_MODEL), jnp.float32),
            pltpu.VMEM((N_DEV - 1, B, SQ_SHARD, D_MODEL), jnp.float32),
            pltpu.SemaphoreType.DMA((N_DEV - 1,)),
            pltpu.SemaphoreType.DMA((N_DEV - 1,)),
            pltpu.SemaphoreType.DMA((N_DEV - 1,)),
            pltpu.SemaphoreType.DMA((N_DEV - 1,)),
        ],
        compiler_params=pltpu.CompilerParams(collective_id=0),
    )(x, Wq, Wk, Wv, Wo)


# baseline (device time: 145665 ns/iter reference)
import jax
import jax.numpy as jnp
from jax import lax
from jax.experimental import pallas as pl
from jax.experimental.pallas import tpu as pltpu

N_DEV = 4
B = 2
SQ_SHARD = 256
SQ = SQ_SHARD * N_DEV
D_MODEL = 768
H_LOC = 4
DH = 64
HD_LOC = H_LOC * DH


def _body(x_ref, wq_ref, wk_ref, wv_ref, wo_ref, out_ref,
          xg_ref, qp_ref, kp_ref, vp_ref, ctx_ref, pout_ref, po_ref, rs_ref,
          ag_send, ag_recv, rs_send, rs_recv):
    my = lax.axis_index("i")
    left = lax.rem(my + N_DEV - 1, N_DEV)
    right = lax.rem(my + 1, N_DEV)

    barrier = pltpu.get_barrier_semaphore()
    pl.semaphore_signal(barrier, inc=1, device_id=(left,),
                        device_id_type=pl.DeviceIdType.MESH)
    pl.semaphore_signal(barrier, inc=1, device_id=(right,),
                        device_id_type=pl.DeviceIdType.MESH)
    pl.semaphore_wait(barrier, 2)

    xg_ref[0] = x_ref[...]
    for h in range(N_DEV - 1):
        rdma = pltpu.make_async_remote_copy(
            src_ref=xg_ref.at[h],
            dst_ref=xg_ref.at[h + 1],
            send_sem=ag_send.at[h],
            recv_sem=ag_recv.at[h],
            device_id=(right,),
            device_id_type=pl.DeviceIdType.MESH,
        )
        rdma.start()
        rdma.wait()

    for r in range(N_DEV):
        blk = lax.rem(my + (N_DEV - r), N_DEV)
        row0 = blk * SQ_SHARD
        for b in range(B):
            xb = xg_ref[r, b]
            qp_ref[b, pl.ds(row0, SQ_SHARD), :] = jnp.dot(
                xb, wq_ref[...], preferred_element_type=jnp.float32)
            kp_ref[b, pl.ds(row0, SQ_SHARD), :] = jnp.dot(
                xb, wk_ref[...], preferred_element_type=jnp.float32)
            vp_ref[b, pl.ds(row0, SQ_SHARD), :] = jnp.dot(
                xb, wv_ref[...], preferred_element_type=jnp.float32)

    rowp = lax.broadcasted_iota(jnp.int32, (SQ, DH), 0).astype(jnp.float32)
    lane = lax.broadcasted_iota(jnp.int32, (SQ, DH), 1)
    k2 = (lane // 2).astype(jnp.float32)
    inv = jnp.exp(k2 * (-2.0 / DH) * jnp.log(10000.0))
    theta = rowp * inv
    cosv = jnp.cos(theta)
    sinv = jnp.sin(theta)
    ar = lax.broadcasted_iota(jnp.int32, (DH, DH), 0)
    ac = lax.broadcasted_iota(jnp.int32, (DH, DH), 1)
    rmat = jnp.where(
        (ar == ac + 1) & (ar % 2 == 1), -1.0,
        jnp.where((ac == ar + 1) & (ar % 2 == 0), 1.0, 0.0),
    ).astype(jnp.float32)

    for b in range(B):
        for h in range(H_LOC):
            q = qp_ref[b, :, pl.ds(h * DH, DH)]
            k = kp_ref[b, :, pl.ds(h * DH, DH)]
            v = vp_ref[b, :, pl.ds(h * DH, DH)]
            qr = q * cosv + jnp.dot(
                q, rmat, preferred_element_type=jnp.float32) * sinv
            kr = k * cosv + jnp.dot(
                k, rmat, preferred_element_type=jnp.float32) * sinv
            s = lax.dot_general(
                qr, kr, (((1,), (1,)), ((), ())),
                preferred_element_type=jnp.float32) * 0.125
            m = jnp.max(s, axis=1, keepdims=True)
            w = jnp.exp(s - m)
            w = w / jnp.sum(w, axis=1, keepdims=True)
            ctx_ref[:, pl.ds(h * DH, DH)] = jnp.dot(
                w, v, preferred_element_type=jnp.float32)
        pout_ref[...] = jnp.dot(
            ctx_ref[...], wo_ref[...], preferred_element_type=jnp.float32)
        for r in range(N_DEV):
            blk = lax.rem(my + (N_DEV - r), N_DEV)
            po_ref[r, b] = pout_ref[pl.ds(blk * SQ_SHARD, SQ_SHARD), :]

    for s in range(N_DEV - 1):
        rdma = pltpu.make_async_remote_copy(
            src_ref=po_ref.at[s + 1],
            dst_ref=rs_ref.at[s],
            send_sem=rs_send.at[s],
            recv_sem=rs_recv.at[s],
            device_id=(right,),
            device_id_type=pl.DeviceIdType.MESH,
        )
        rdma.start()
        rdma.wait()
        dst = (s + 2) % N_DEV
        po_ref[dst] = po_ref[dst] + rs_ref[s]

    out_ref[...] = po_ref[0]


def kernel(x, Wq, Wk, Wv, Wo):
    return pl.pallas_call(
        _body,
        out_shape=jax.ShapeDtypeStruct((B, SQ_SHARD, D_MODEL), jnp.float32),
        in_specs=[pl.BlockSpec(memory_space=pltpu.VMEM)] * 5,
        out_specs=pl.BlockSpec(memory_space=pltpu.VMEM),
        scratch_shapes=[
            pltpu.VMEM((N_DEV, B, SQ_SHARD, D_MODEL), jnp.float32),
            pltpu.VMEM((B, SQ, HD_LOC), jnp.float32),
            pltpu.VMEM((B, SQ, HD_LOC), jnp.float32),
            pltpu.VMEM((B, SQ, HD_LOC), jnp.float32),
            pltpu.VMEM((SQ, HD_LOC), jnp.float32),
            pltpu.VMEM((SQ, D_MODEL), jnp.float32),
            pltpu.VMEM((N_DEV, B, SQ_SHARD, D_MODEL), jnp.float32),
            pltpu.VMEM((N_DEV - 1, B, SQ_SHARD, D_MODEL), jnp.float32),
            pltpu.SemaphoreType.DMA((N_DEV - 1,)),
            pltpu.SemaphoreType.DMA((N_DEV - 1,)),
            pltpu.SemaphoreType.DMA((N_DEV - 1,)),
            pltpu.SemaphoreType.DMA((N_DEV - 1,)),
        ],
        compiler_params=pltpu.CompilerParams(
            collective_id=0, vmem_limit_bytes=100 * 1024 * 1024),
    )(x, Wq, Wk, Wv, Wo)


# device time: 99224 ns/iter; 1.4680x vs baseline; 1.4680x over previous
import jax
import jax.numpy as jnp
from jax import lax
from jax.experimental import pallas as pl
from jax.experimental.pallas import tpu as pltpu

N_DEV = 4
B = 2
SQ_SHARD = 256
SQ = SQ_SHARD * N_DEV
D_MODEL = 768
H_LOC = 4
DH = 64
HD_LOC = H_LOC * DH

CW1, CCW1, CW2 = 0, 1, 2


def _body(x_ref, wq_ref, wk_ref, wv_ref, wo_ref, out_ref,
          xg_ref, qp_ref, kp_ref, vp_ref, ctx_ref, po_ref, rs_ref,
          ag_send, ag_recv, rs_send, rs_recv):
    my = lax.axis_index("i")
    left = lax.rem(my + N_DEV - 1, N_DEV)
    right = lax.rem(my + 1, N_DEV)

    barrier = pltpu.get_barrier_semaphore()
    pl.semaphore_signal(barrier, inc=1, device_id=(left,),
                        device_id_type=pl.DeviceIdType.MESH)
    pl.semaphore_signal(barrier, inc=1, device_id=(right,),
                        device_id_type=pl.DeviceIdType.MESH)
    pl.semaphore_wait(barrier, 2)

    def ag_rdma(idx, src_slot, dst_slot, tgt):
        return pltpu.make_async_remote_copy(
            src_ref=xg_ref.at[src_slot], dst_ref=xg_ref.at[dst_slot],
            send_sem=ag_send.at[idx], recv_sem=ag_recv.at[idx],
            device_id=(tgt,), device_id_type=pl.DeviceIdType.MESH)

    def proj(r):
        blk = lax.rem(my + (N_DEV - r), N_DEV)
        row0 = blk * SQ_SHARD
        for b in range(B):
            xb = xg_ref[r, b]
            qp_ref[b, pl.ds(row0, SQ_SHARD), :] = jnp.dot(
                xb, wq_ref[...], preferred_element_type=jnp.float32)
            kp_ref[b, pl.ds(row0, SQ_SHARD), :] = jnp.dot(
                xb, wk_ref[...], preferred_element_type=jnp.float32)
            vp_ref[b, pl.ds(row0, SQ_SHARD), :] = jnp.dot(
                xb, wv_ref[...], preferred_element_type=jnp.float32)

    xg_ref[0] = x_ref[...]
    cw1 = ag_rdma(CW1, 0, 1, right)
    ccw1 = ag_rdma(CCW1, 0, 3, left)
    cw1.start()
    ccw1.start()
    proj(0)
    cw1.wait_recv()
    cw2 = ag_rdma(CW2, 1, 2, right)
    cw2.start()
    ccw1.wait_recv()
    proj(1)
    proj(3)
    cw2.wait_recv()
    proj(2)

    rowp = lax.broadcasted_iota(jnp.int32, (SQ, HD_LOC), 0).astype(jnp.float32)
    lane = lax.broadcasted_iota(jnp.int32, (SQ, HD_LOC), 1)
    k2 = ((lane % DH) // 2).astype(jnp.float32)
    theta = rowp * jnp.exp(k2 * (-2.0 / DH) * jnp.log(10000.0))
    cosv = jnp.cos(theta)
    sinv = jnp.sin(theta)
    ar = lax.broadcasted_iota(jnp.int32, (HD_LOC, HD_LOC), 0)
    ac = lax.broadcasted_iota(jnp.int32, (HD_LOC, HD_LOC), 1)
    rmat = jnp.where(
        (ar == ac + 1) & (ar % 2 == 1), -1.0,
        jnp.where((ac == ar + 1) & (ar % 2 == 0), 1.0, 0.0),
    ).astype(jnp.float32)

    for b in range(B):
        q = qp_ref[b]
        qp_ref[b] = q * cosv + jnp.dot(
            q, rmat, preferred_element_type=jnp.float32) * sinv
        k = kp_ref[b]
        kp_ref[b] = k * cosv + jnp.dot(
            k, rmat, preferred_element_type=jnp.float32) * sinv

    def attn_block(r):
        blk = lax.rem(my + (N_DEV - r), N_DEV)
        row0 = blk * SQ_SHARD
        for b in range(B):
            for h in range(H_LOC):
                q = qp_ref[b, pl.ds(row0, SQ_SHARD), pl.ds(h * DH, DH)]
                k = kp_ref[b, :, pl.ds(h * DH, DH)]
                v = vp_ref[b, :, pl.ds(h * DH, DH)]
                s = lax.dot_general(
                    q, k, (((1,), (1,)), ((), ())),
                    preferred_element_type=jnp.float32) * 0.125
                m = jnp.max(s, axis=1, keepdims=True)
                w = jnp.exp(s - m)
                w = w / jnp.sum(w, axis=1, keepdims=True)
                ctx_ref[:, pl.ds(h * DH, DH)] = jnp.dot(
                    w, v, preferred_element_type=jnp.float32)
            po_ref[r, b] = jnp.dot(
                ctx_ref[...], wo_ref[...], preferred_element_type=jnp.float32)

    def rs_rdma(idx, src_slot, tgt):
        return pltpu.make_async_remote_copy(
            src_ref=po_ref.at[src_slot], dst_ref=rs_ref.at[idx],
            send_sem=rs_send.at[idx], recv_sem=rs_recv.at[idx],
            device_id=(tgt,), device_id_type=pl.DeviceIdType.MESH)

    attn_block(1)
    attn_block(2)
    r_ccw1 = rs_rdma(CCW1, 1, left)
    r_cw1 = rs_rdma(CW1, 2, right)
    r_ccw1.start()
    r_cw1.start()
    attn_block(3)
    r_cw1.wait_recv()
    po_ref[3] = po_ref[3] + rs_ref[CW1]
    r_cw2 = rs_rdma(CW2, 3, right)
    r_cw2.start()
    attn_block(0)
    r_ccw1.wait_recv()
    r_cw2.wait_recv()
    out_ref[...] = po_ref[0] + rs_ref[CCW1] + rs_ref[CW2]

    cw1.wait_send()
    ccw1.wait_send()
    cw2.wait_send()
    r_ccw1.wait_send()
    r_cw1.wait_send()
    r_cw2.wait_send()


def kernel(x, Wq, Wk, Wv, Wo):
    return pl.pallas_call(
        _body,
        out_shape=jax.ShapeDtypeStruct((B, SQ_SHARD, D_MODEL), jnp.float32),
        in_specs=[pl.BlockSpec(memory_space=pltpu.VMEM)] * 5,
        out_specs=pl.BlockSpec(memory_space=pltpu.VMEM),
        scratch_shapes=[
            pltpu.VMEM((N_DEV, B, SQ_SHARD, D_MODEL), jnp.float32),
            pltpu.VMEM((B, SQ, HD_LOC), jnp.float32),
            pltpu.VMEM((B, SQ, HD_LOC), jnp.float32),
            pltpu.VMEM((B, SQ, HD_LOC), jnp.float32),
            pltpu.VMEM((SQ_SHARD, HD_LOC), jnp.float32),
            pltpu.VMEM((N_DEV, B, SQ_SHARD, D_MODEL), jnp.float32),
            pltpu.VMEM((N_DEV - 1, B, SQ_SHARD, D_MODEL), jnp.float32),
            pltpu.SemaphoreType.DMA((N_DEV - 1,)),
            pltpu.SemaphoreType.DMA((N_DEV - 1,)),
            pltpu.SemaphoreType.DMA((N_DEV - 1,)),
            pltpu.SemaphoreType.DMA((N_DEV - 1,)),
        ],
        compiler_params=pltpu.CompilerParams(
            collective_id=0, vmem_limit_bytes=100 * 1024 * 1024),
    )(x, Wq, Wk, Wv, Wo)


# device time: 78662 ns/iter; 1.8518x vs baseline; 1.2614x over previous
import jax
import jax.numpy as jnp
from jax import lax
from jax.experimental import pallas as pl
from jax.experimental.pallas import tpu as pltpu

N_DEV = 4
B = 2
SQ_SHARD = 256
SQ = SQ_SHARD * N_DEV
D_MODEL = 768
H_LOC = 4
DH = 64
HD_LOC = H_LOC * DH

CW1, CCW1, CW2, CCW2 = 0, 1, 2, 3


def _body(x_ref, wq_ref, wk_ref, wv_ref, wo_ref, out_ref,
          xg_ref, qp_ref, kp_ref, vp_ref, ctx_ref, po_ref, rs_ref,
          cos_ref, sin_ref, ag_send, ag_recv, rs_send, rs_recv):
    my = lax.axis_index("i")
    left = lax.rem(my + N_DEV - 1, N_DEV)
    right = lax.rem(my + 1, N_DEV)

    barrier = pltpu.get_barrier_semaphore()
    pl.semaphore_signal(barrier, inc=1, device_id=(left,),
                        device_id_type=pl.DeviceIdType.MESH)
    pl.semaphore_signal(barrier, inc=1, device_id=(right,),
                        device_id_type=pl.DeviceIdType.MESH)
    pl.semaphore_wait(barrier, 2)

    rowp = lax.broadcasted_iota(jnp.int32, (SQ, HD_LOC), 0).astype(jnp.float32)
    lane = lax.broadcasted_iota(jnp.int32, (SQ, HD_LOC), 1)
    k2 = ((lane % DH) // 2).astype(jnp.float32)
    theta = rowp * jnp.exp(k2 * (-2.0 / DH) * jnp.log(10000.0))
    cos_ref[...] = jnp.cos(theta)
    sin_ref[...] = jnp.sin(theta)
    ar = lax.broadcasted_iota(jnp.int32, (HD_LOC, HD_LOC), 0)
    ac = lax.broadcasted_iota(jnp.int32, (HD_LOC, HD_LOC), 1)
    rmat = jnp.where(
        (ar == ac + 1) & (ar % 2 == 1), -1.0,
        jnp.where((ac == ar + 1) & (ar % 2 == 0), 1.0, 0.0),
    ).astype(jnp.float32)

    def ag_rdma(idx, src, dst, tgt):
        return pltpu.make_async_remote_copy(
            src_ref=src, dst_ref=dst,
            send_sem=ag_send.at[idx], recv_sem=ag_recv.at[idx],
            device_id=(tgt,), device_id_type=pl.DeviceIdType.MESH)

    def proj(r):
        blk = lax.rem(my + (N_DEV - r), N_DEV)
        row0 = blk * SQ_SHARD
        cos_b = cos_ref[pl.ds(row0, SQ_SHARD), :]
        sin_b = sin_ref[pl.ds(row0, SQ_SHARD), :]
        for b in range(B):
            xb = xg_ref[r, b]
            q = jnp.dot(xb, wq_ref[...], preferred_element_type=jnp.float32)
            qp_ref[b, pl.ds(row0, SQ_SHARD), :] = q * cos_b + jnp.dot(
                q, rmat, preferred_element_type=jnp.float32) * sin_b
            k = jnp.dot(xb, wk_ref[...], preferred_element_type=jnp.float32)
            kp_ref[b, pl.ds(row0, SQ_SHARD), :] = k * cos_b + jnp.dot(
                k, rmat, preferred_element_type=jnp.float32) * sin_b
            vp_ref[b, pl.ds(row0, SQ_SHARD), :] = jnp.dot(
                xb, wv_ref[...], preferred_element_type=jnp.float32)

    xg_ref[0] = x_ref[...]
    cw1 = ag_rdma(CW1, xg_ref.at[0], xg_ref.at[1], right)
    ccw1 = ag_rdma(CCW1, xg_ref.at[0], xg_ref.at[3], left)
    cw1.start()
    ccw1.start()
    proj(0)
    cw1.wait_recv()
    cw2 = ag_rdma(CW2, xg_ref.at[1, 0], xg_ref.at[2, 0], right)
    cw2.start()
    ccw1.wait_recv()
    ccw2 = ag_rdma(CCW2, xg_ref.at[3, 1], xg_ref.at[2, 1], left)
    ccw2.start()
    proj(1)
    proj(3)
    cw2.wait_recv()
    ccw2.wait_recv()
    proj(2)

    def attn_block(r, b):
        blk = lax.rem(my + (N_DEV - r), N_DEV)
        row0 = blk * SQ_SHARD
        for h in range(H_LOC):
            q = qp_ref[b, pl.ds(row0, SQ_SHARD), pl.ds(h * DH, DH)]
            k = kp_ref[b, :, pl.ds(h * DH, DH)]
            v = vp_ref[b, :, pl.ds(h * DH, DH)]
            s = lax.dot_general(
                q, k, (((1,), (1,)), ((), ())),
                preferred_element_type=jnp.float32) * 0.125
            m = jnp.max(s, axis=1, keepdims=True)
            w = jnp.exp(s - m)
            w = w / jnp.sum(w, axis=1, keepdims=True)
            ctx_ref[:, pl.ds(h * DH, DH)] = jnp.dot(
                w, v, preferred_element_type=jnp.float32)
        po_ref[r, b] = jnp.dot(
            ctx_ref[...], wo_ref[...], preferred_element_type=jnp.float32)

    def rs_rdma(idx, src, dst, tgt):
        return pltpu.make_async_remote_copy(
            src_ref=src, dst_ref=dst,
            send_sem=rs_send.at[idx], recv_sem=rs_recv.at[idx],
            device_id=(tgt,), device_id_type=pl.DeviceIdType.MESH)

    attn_block(2, 0)
    r_cw1 = rs_rdma(CW1, po_ref.at[2, 0], rs_ref.at[CW1, 0], right)
    r_cw1.start()
    attn_block(2, 1)
    r_ccw1 = rs_rdma(CCW1, po_ref.at[2, 1], rs_ref.at[CCW1, 1], left)
    r_ccw1.start()
    attn_block(3, 0)
    attn_block(3, 1)
    attn_block(1, 0)
    attn_block(1, 1)
    r_cw1.wait_recv()
    po_ref[3, 0] = po_ref[3, 0] + rs_ref[CW1, 0]
    r_cw2 = rs_rdma(CW2, po_ref.at[3], rs_ref.at[CW2], right)
    r_cw2.start()
    r_ccw1.wait_recv()
    po_ref[1, 1] = po_ref[1, 1] + rs_ref[CCW1, 1]
    r_ccw2 = rs_rdma(CCW2, po_ref.at[1], rs_ref.at[CCW2], left)
    r_ccw2.start()
    attn_block(0, 0)
    attn_block(0, 1)
    r_cw2.wait_recv()
    r_ccw2.wait_recv()
    out_ref[...] = po_ref[0] + rs_ref[CW2] + rs_ref[CCW2]

    for d in (cw1, ccw1, cw2, ccw2, r_cw1, r_ccw1, r_cw2, r_ccw2):
        d.wait_send()


def kernel(x, Wq, Wk, Wv, Wo):
    return pl.pallas_call(
        _body,
        out_shape=jax.ShapeDtypeStruct((B, SQ_SHARD, D_MODEL), jnp.float32),
        in_specs=[pl.BlockSpec(memory_space=pltpu.VMEM)] * 5,
        out_specs=pl.BlockSpec(memory_space=pltpu.VMEM),
        scratch_shapes=[
            pltpu.VMEM((N_DEV, B, SQ_SHARD, D_MODEL), jnp.float32),
            pltpu.VMEM((B, SQ, HD_LOC), jnp.float32),
            pltpu.VMEM((B, SQ, HD_LOC), jnp.float32),
            pltpu.VMEM((B, SQ, HD_LOC), jnp.float32),
            pltpu.VMEM((SQ_SHARD, HD_LOC), jnp.float32),
            pltpu.VMEM((N_DEV, B, SQ_SHARD, D_MODEL), jnp.float32),
            pltpu.VMEM((N_DEV, B, SQ_SHARD, D_MODEL), jnp.float32),
            pltpu.VMEM((SQ, HD_LOC), jnp.float32),
            pltpu.VMEM((SQ, HD_LOC), jnp.float32),
            pltpu.SemaphoreType.DMA((N_DEV,)),
            pltpu.SemaphoreType.DMA((N_DEV,)),
            pltpu.SemaphoreType.DMA((N_DEV,)),
            pltpu.SemaphoreType.DMA((N_DEV,)),
        ],
        compiler_params=pltpu.CompilerParams(
            collective_id=0, vmem_limit_bytes=100 * 1024 * 1024),
    )(x, Wq, Wk, Wv, Wo)


# device time: 74467 ns/iter; 1.9561x vs baseline; 1.0563x over previous
import jax
import jax.numpy as jnp
from jax import lax
from jax.experimental import pallas as pl
from jax.experimental.pallas import tpu as pltpu

N_DEV = 4
B = 2
SQ_SHARD = 256
SQ = SQ_SHARD * N_DEV
D_MODEL = 768
H_LOC = 4
DH = 64
HD_LOC = H_LOC * DH

CW1, CCW1, CW2, CCW2 = 0, 1, 2, 3
FROM_LEFT, FROM_RIGHT, FROM_DIAG = 0, 1, 2


def _body(x_ref, wq_ref, wk_ref, wv_ref, wo_ref, out_ref,
          xg_ref, qp_ref, kp_ref, vp_ref, ctxa_ref, wo_all_ref, rx_ref,
          cos_ref, sin_ref,
          ag_send, ag_recv, wo_send, wo_recv, a2a_send, a2a_recv):
    my = lax.axis_index("i")
    left = lax.rem(my + N_DEV - 1, N_DEV)
    right = lax.rem(my + 1, N_DEV)
    diag = lax.rem(my + 2, N_DEV)

    barrier = pltpu.get_barrier_semaphore()
    for nbr in (left, right, diag):
        pl.semaphore_signal(barrier, inc=1, device_id=(nbr,),
                            device_id_type=pl.DeviceIdType.MESH)
    pl.semaphore_wait(barrier, 3)

    rowp = lax.broadcasted_iota(jnp.int32, (SQ, HD_LOC), 0).astype(jnp.float32)
    lane = lax.broadcasted_iota(jnp.int32, (SQ, HD_LOC), 1)
    k2 = ((lane % DH) // 2).astype(jnp.float32)
    theta = rowp * jnp.exp(k2 * (-2.0 / DH) * jnp.log(10000.0))
    cos_ref[...] = jnp.cos(theta)
    sin_ref[...] = jnp.sin(theta)
    ar = lax.broadcasted_iota(jnp.int32, (HD_LOC, HD_LOC), 0)
    ac = lax.broadcasted_iota(jnp.int32, (HD_LOC, HD_LOC), 1)
    rmat = jnp.where(
        (ar == ac + 1) & (ar % 2 == 1), -1.0,
        jnp.where((ac == ar + 1) & (ar % 2 == 0), 1.0, 0.0),
    ).astype(jnp.float32)

    def ag_rdma(idx, src, dst, tgt):
        return pltpu.make_async_remote_copy(
            src_ref=src, dst_ref=dst,
            send_sem=ag_send.at[idx], recv_sem=ag_recv.at[idx],
            device_id=(tgt,), device_id_type=pl.DeviceIdType.MESH)

    def wo_rdma(idx, src_slot, dst_slot, tgt):
        return pltpu.make_async_remote_copy(
            src_ref=wo_all_ref.at[src_slot], dst_ref=wo_all_ref.at[dst_slot],
            send_sem=wo_send.at[idx], recv_sem=wo_recv.at[idx],
            device_id=(tgt,), device_id_type=pl.DeviceIdType.MESH)

    def proj(r):
        blk = lax.rem(my + (N_DEV - r), N_DEV)
        row0 = blk * SQ_SHARD
        cos_b = cos_ref[pl.ds(row0, SQ_SHARD), :]
        sin_b = sin_ref[pl.ds(row0, SQ_SHARD), :]
        for b in range(B):
            xb = xg_ref[r, b]
            q = jnp.dot(xb, wq_ref[...], preferred_element_type=jnp.float32)
            qp_ref[b, pl.ds(row0, SQ_SHARD), :] = q * cos_b + jnp.dot(
                q, rmat, preferred_element_type=jnp.float32) * sin_b
            k = jnp.dot(xb, wk_ref[...], preferred_element_type=jnp.float32)
            kp_ref[b, pl.ds(row0, SQ_SHARD), :] = k * cos_b + jnp.dot(
                k, rmat, preferred_element_type=jnp.float32) * sin_b
            vp_ref[b, pl.ds(row0, SQ_SHARD), :] = jnp.dot(
                xb, wv_ref[...], preferred_element_type=jnp.float32)

    xg_ref[0] = x_ref[...]
    wo_all_ref[0] = wo_ref[...]
    cw1 = ag_rdma(CW1, xg_ref.at[0], xg_ref.at[1], right)
    ccw1 = ag_rdma(CCW1, xg_ref.at[0], xg_ref.at[3], left)
    cw1.start()
    ccw1.start()
    w_cw1 = wo_rdma(CW1, 0, 1, right)
    w_ccw1 = wo_rdma(CCW1, 0, 3, left)
    w_cw1.start()
    w_ccw1.start()
    proj(0)
    cw1.wait_recv()
    cw2 = ag_rdma(CW2, xg_ref.at[1, 0], xg_ref.at[2, 0], right)
    cw2.start()
    ccw1.wait_recv()
    ccw2 = ag_rdma(CCW2, xg_ref.at[3, 1], xg_ref.at[2, 1], left)
    ccw2.start()
    proj(1)
    proj(3)
    w_cw1.wait_recv()
    w_cw2 = wo_rdma(CW2, 1, 2, right)
    w_cw2.start()
    cw2.wait_recv()
    ccw2.wait_recv()
    proj(2)

    def attn_block(r, b):
        blk = lax.rem(my + (N_DEV - r), N_DEV)
        row0 = blk * SQ_SHARD
        for h in range(H_LOC):
            q = qp_ref[b, pl.ds(row0, SQ_SHARD), pl.ds(h * DH, DH)]
            k = kp_ref[b, :, pl.ds(h * DH, DH)]
            v = vp_ref[b, :, pl.ds(h * DH, DH)]
            s = lax.dot_general(
                q, k, (((1,), (1,)), ((), ())),
                preferred_element_type=jnp.float32) * 0.125
            m = jnp.max(s, axis=1, keepdims=True)
            w = jnp.exp(s - m)
            w = w / jnp.sum(w, axis=1, keepdims=True)
            ctxa_ref[r, b, :, pl.ds(h * DH, DH)] = jnp.dot(
                w, v, preferred_element_type=jnp.float32)

    def a2a_rdma(idx, src_slot, rx_slot, tgt):
        return pltpu.make_async_remote_copy(
            src_ref=ctxa_ref.at[src_slot], dst_ref=rx_ref.at[rx_slot],
            send_sem=a2a_send.at[idx], recv_sem=a2a_recv.at[idx],
            device_id=(tgt,), device_id_type=pl.DeviceIdType.MESH)

    attn_block(2, 0)
    attn_block(2, 1)
    s_diag = a2a_rdma(FROM_DIAG, 2, FROM_DIAG, diag)
    s_diag.start()
    attn_block(1, 0)
    attn_block(1, 1)
    s_left = a2a_rdma(FROM_RIGHT, 1, FROM_RIGHT, left)
    s_left.start()
    attn_block(3, 0)
    attn_block(3, 1)
    s_right = a2a_rdma(FROM_LEFT, 3, FROM_LEFT, right)
    s_right.start()
    attn_block(0, 0)
    attn_block(0, 1)

    w_ccw1.wait_recv()
    w_cw2.wait_recv()
    for b in range(B):
        out_ref[b] = jnp.dot(ctxa_ref[0, b], wo_all_ref[0],
                             preferred_element_type=jnp.float32)

    for rx_slot, wo_slot in ((FROM_LEFT, 1), (FROM_DIAG, 2),
                             (FROM_RIGHT, 3)):
        rcv = pltpu.make_async_remote_copy(
            src_ref=ctxa_ref.at[0], dst_ref=rx_ref.at[rx_slot],
            send_sem=a2a_send.at[rx_slot], recv_sem=a2a_recv.at[rx_slot],
            device_id=(my,), device_id_type=pl.DeviceIdType.MESH)
        rcv.wait_recv()
        for b in range(B):
            out_ref[b] = out_ref[b] + jnp.dot(
                rx_ref[rx_slot, b], wo_all_ref[wo_slot],
                preferred_element_type=jnp.float32)

    for d in (cw1, ccw1, cw2, ccw2, w_cw1, w_ccw1, w_cw2,
              s_diag, s_left, s_right):
        d.wait_send()


def kernel(x, Wq, Wk, Wv, Wo):
    return pl.pallas_call(
        _body,
        out_shape=jax.ShapeDtypeStruct((B, SQ_SHARD, D_MODEL), jnp.float32),
        in_specs=[pl.BlockSpec(memory_space=pltpu.VMEM)] * 5,
        out_specs=pl.BlockSpec(memory_space=pltpu.VMEM),
        scratch_shapes=[
            pltpu.VMEM((N_DEV, B, SQ_SHARD, D_MODEL), jnp.float32),
            pltpu.VMEM((B, SQ, HD_LOC), jnp.float32),
            pltpu.VMEM((B, SQ, HD_LOC), jnp.float32),
            pltpu.VMEM((B, SQ, HD_LOC), jnp.float32),
            pltpu.VMEM((N_DEV, B, SQ_SHARD, HD_LOC), jnp.float32),
            pltpu.VMEM((N_DEV, HD_LOC, D_MODEL), jnp.float32),
            pltpu.VMEM((3, B, SQ_SHARD, HD_LOC), jnp.float32),
            pltpu.VMEM((SQ, HD_LOC), jnp.float32),
            pltpu.VMEM((SQ, HD_LOC), jnp.float32),
            pltpu.SemaphoreType.DMA((N_DEV,)),
            pltpu.SemaphoreType.DMA((N_DEV,)),
            pltpu.SemaphoreType.DMA((3,)),
            pltpu.SemaphoreType.DMA((3,)),
            pltpu.SemaphoreType.DMA((3,)),
            pltpu.SemaphoreType.DMA((3,)),
        ],
        compiler_params=pltpu.CompilerParams(
            collective_id=0, vmem_limit_bytes=100 * 1024 * 1024),
    )(x, Wq, Wk, Wv, Wo)


# device time: 70396 ns/iter; 2.0692x vs baseline; 1.0578x over previous
import jax
import jax.numpy as jnp
from jax import lax
from jax.experimental import pallas as pl
from jax.experimental.pallas import tpu as pltpu

N_DEV = 4
B = 2
SQ_SHARD = 256
SQ = SQ_SHARD * N_DEV
D_MODEL = 768
H_LOC = 4
DH = 64
HD_LOC = H_LOC * DH

CW1, CCW1, CW2, CCW2 = 0, 1, 2, 3
FROM_LEFT, FROM_RIGHT, FROM_DIAG = 0, 1, 2


def _body(x_ref, wq_ref, wk_ref, wv_ref, wo_ref, out_ref,
          xg_ref, qp_ref, kp_ref, vp_ref, ctxa_ref, wo_all_ref, rx_ref,
          cos_ref, sin_ref,
          ag_send, ag_recv, wo_send, wo_recv, a2a_send, a2a_recv):
    my = lax.axis_index("i")
    left = lax.rem(my + N_DEV - 1, N_DEV)
    right = lax.rem(my + 1, N_DEV)
    diag = lax.rem(my + 2, N_DEV)

    barrier = pltpu.get_barrier_semaphore()
    for nbr in (left, right, diag):
        pl.semaphore_signal(barrier, inc=1, device_id=(nbr,),
                            device_id_type=pl.DeviceIdType.MESH)
    pl.semaphore_wait(barrier, 3)

    def ag_rdma(idx, src, dst, tgt):
        return pltpu.make_async_remote_copy(
            src_ref=src, dst_ref=dst,
            send_sem=ag_send.at[idx], recv_sem=ag_recv.at[idx],
            device_id=(tgt,), device_id_type=pl.DeviceIdType.MESH)

    def wo_rdma(idx, src, dst_slot, tgt):
        return pltpu.make_async_remote_copy(
            src_ref=src, dst_ref=wo_all_ref.at[dst_slot],
            send_sem=wo_send.at[idx], recv_sem=wo_recv.at[idx],
            device_id=(tgt,), device_id_type=pl.DeviceIdType.MESH)

    def proj(r):
        blk = lax.rem(my + (N_DEV - r), N_DEV)
        row0 = blk * SQ_SHARD
        cos_b = cos_ref[pl.ds(row0, SQ_SHARD), :]
        sin_b = sin_ref[pl.ds(row0, SQ_SHARD), :]
        for b in range(B):
            xb = x_ref[b] if r == 0 else xg_ref[r, b]
            q = jnp.dot(xb, wq_ref[...], preferred_element_type=jnp.float32)
            qp_ref[b, pl.ds(row0, SQ_SHARD), :] = q * cos_b + jnp.dot(
                q, rmat, preferred_element_type=jnp.float32) * sin_b
            k = jnp.dot(xb, wk_ref[...], preferred_element_type=jnp.float32)
            kp_ref[b, pl.ds(row0, SQ_SHARD), :] = k * cos_b + jnp.dot(
                k, rmat, preferred_element_type=jnp.float32) * sin_b
            vp_ref[b, pl.ds(row0, SQ_SHARD), :] = jnp.dot(
                xb, wv_ref[...], preferred_element_type=jnp.float32)

    cw1 = ag_rdma(CW1, x_ref, xg_ref.at[1], right)
    ccw1 = ag_rdma(CCW1, x_ref, xg_ref.at[3], left)
    cw1.start()
    ccw1.start()
    w_cw1 = wo_rdma(CW1, wo_ref, 1, right)
    w_ccw1 = wo_rdma(CCW1, wo_ref, 3, left)
    w_cw1.start()
    w_ccw1.start()

    rowp = lax.broadcasted_iota(jnp.int32, (SQ, HD_LOC), 0).astype(jnp.float32)
    lane = lax.broadcasted_iota(jnp.int32, (SQ, HD_LOC), 1)
    k2 = ((lane % DH) // 2).astype(jnp.float32)
    theta = rowp * jnp.exp(k2 * (-2.0 / DH) * jnp.log(10000.0))
    cos_ref[...] = jnp.cos(theta)
    sin_ref[...] = jnp.sin(theta)
    ar = lax.broadcasted_iota(jnp.int32, (HD_LOC, HD_LOC), 0)
    ac = lax.broadcasted_iota(jnp.int32, (HD_LOC, HD_LOC), 1)
    rmat = jnp.where(
        (ar == ac + 1) & (ar % 2 == 1), -1.0,
        jnp.where((ac == ar + 1) & (ar % 2 == 0), 1.0, 0.0),
    ).astype(jnp.float32)

    proj(0)
    cw1.wait_recv()
    cw2 = ag_rdma(CW2, xg_ref.at[1, 0], xg_ref.at[2, 0], right)
    cw2.start()
    ccw1.wait_recv()
    ccw2 = ag_rdma(CCW2, xg_ref.at[3, 1], xg_ref.at[2, 1], left)
    ccw2.start()
    proj(1)
    proj(3)
    w_cw1.wait_recv()
    w_cw2 = wo_rdma(CW2, wo_all_ref.at[1], 2, right)
    w_cw2.start()
    cw2.wait_recv()
    ccw2.wait_recv()
    proj(2)

    def attn_block(r, b):
        blk = lax.rem(my + (N_DEV - r), N_DEV)
        row0 = blk * SQ_SHARD
        for h in range(H_LOC):
            q = qp_ref[b, pl.ds(row0, SQ_SHARD), pl.ds(h * DH, DH)]
            k = kp_ref[b, :, pl.ds(h * DH, DH)]
            v = vp_ref[b, :, pl.ds(h * DH, DH)]
            s = lax.dot_general(
                q, k, (((1,), (1,)), ((), ())),
                preferred_element_type=jnp.float32) * 0.125
            m = jnp.max(s, axis=1, keepdims=True)
            w = jnp.exp(s - m)
            w = w / jnp.sum(w, axis=1, keepdims=True)
            ctxa_ref[r, b, :, pl.ds(h * DH, DH)] = jnp.dot(
                w, v, preferred_element_type=jnp.float32)

    def a2a_rdma(idx, src_slot, rx_slot, tgt):
        return pltpu.make_async_remote_copy(
            src_ref=ctxa_ref.at[src_slot], dst_ref=rx_ref.at[rx_slot],
            send_sem=a2a_send.at[idx], recv_sem=a2a_recv.at[idx],
            device_id=(tgt,), device_id_type=pl.DeviceIdType.MESH)

    attn_block(2, 0)
    attn_block(2, 1)
    s_diag = a2a_rdma(FROM_DIAG, 2, FROM_DIAG, diag)
    s_diag.start()
    attn_block(1, 0)
    attn_block(1, 1)
    s_left = a2a_rdma(FROM_RIGHT, 1, FROM_RIGHT, left)
    s_left.start()
    attn_block(3, 0)
    attn_block(3, 1)
    s_right = a2a_rdma(FROM_LEFT, 3, FROM_LEFT, right)
    s_right.start()
    attn_block(0, 0)
    attn_block(0, 1)

    w_ccw1.wait_recv()
    w_cw2.wait_recv()
    for b in range(B):
        out_ref[b] = jnp.dot(ctxa_ref[0, b], wo_ref[...],
                             preferred_element_type=jnp.float32)

    for rx_slot, wo_slot in ((FROM_LEFT, 1), (FROM_DIAG, 2),
                             (FROM_RIGHT, 3)):
        rcv = pltpu.make_async_remote_copy(
            src_ref=ctxa_ref.at[0], dst_ref=rx_ref.at[rx_slot],
            send_sem=a2a_send.at[rx_slot], recv_sem=a2a_recv.at[rx_slot],
            device_id=(my,), device_id_type=pl.DeviceIdType.MESH)
        rcv.wait_recv()
        for b in range(B):
            out_ref[b] = out_ref[b] + jnp.dot(
                rx_ref[rx_slot, b], wo_all_ref[wo_slot],
                preferred_element_type=jnp.float32)

    for d in (cw1, ccw1, cw2, ccw2, w_cw1, w_ccw1, w_cw2,
              s_diag, s_left, s_right):
        d.wait_send()


def kernel(x, Wq, Wk, Wv, Wo):
    return pl.pallas_call(
        _body,
        out_shape=jax.ShapeDtypeStruct((B, SQ_SHARD, D_MODEL), jnp.float32),
        in_specs=[pl.BlockSpec(memory_space=pltpu.VMEM)] * 5,
        out_specs=pl.BlockSpec(memory_space=pltpu.VMEM),
        scratch_shapes=[
            pltpu.VMEM((N_DEV, B, SQ_SHARD, D_MODEL), jnp.float32),
            pltpu.VMEM((B, SQ, HD_LOC), jnp.float32),
            pltpu.VMEM((B, SQ, HD_LOC), jnp.float32),
            pltpu.VMEM((B, SQ, HD_LOC), jnp.float32),
            pltpu.VMEM((N_DEV, B, SQ_SHARD, HD_LOC), jnp.float32),
            pltpu.VMEM((N_DEV, HD_LOC, D_MODEL), jnp.float32),
            pltpu.VMEM((3, B, SQ_SHARD, HD_LOC), jnp.float32),
            pltpu.VMEM((SQ, HD_LOC), jnp.float32),
            pltpu.VMEM((SQ, HD_LOC), jnp.float32),
            pltpu.SemaphoreType.DMA((N_DEV,)),
            pltpu.SemaphoreType.DMA((N_DEV,)),
            pltpu.SemaphoreType.DMA((3,)),
            pltpu.SemaphoreType.DMA((3,)),
            pltpu.SemaphoreType.DMA((3,)),
            pltpu.SemaphoreType.DMA((3,)),
        ],
        compiler_params=pltpu.CompilerParams(
            collective_id=0, vmem_limit_bytes=100 * 1024 * 1024),
    )(x, Wq, Wk, Wv, Wo)


# device time: 48411 ns/iter; 3.0089x vs baseline; 1.4541x over previous
import jax
import jax.numpy as jnp
from jax import lax
from jax.experimental import pallas as pl
from jax.experimental.pallas import tpu as pltpu

N_DEV = 4
B = 2
SQ_SHARD = 256
SQ = SQ_SHARD * N_DEV
D_MODEL = 768
H_LOC = 4
DH = 64
HD_LOC = H_LOC * DH

BF16 = jnp.bfloat16
F32 = jnp.float32

CW1, CCW1, CW2, CCW2 = 0, 1, 2, 3
FROM_LEFT, FROM_RIGHT, FROM_DIAG = 0, 1, 2


def _body(x_ref, wq_ref, wk_ref, wv_ref, wo_ref, out_ref,
          x16_ref, xg_ref, wqkv_ref, wo16_ref, wo_all_ref,
          qp_ref, kp_ref, vp_ref, ctxa_ref, rx_ref, cos_ref, sin_ref,
          ag_send, ag_recv, wo_send, wo_recv, a2a_send, a2a_recv):
    my = lax.axis_index("i")
    left = lax.rem(my + N_DEV - 1, N_DEV)
    right = lax.rem(my + 1, N_DEV)
    diag = lax.rem(my + 2, N_DEV)

    barrier = pltpu.get_barrier_semaphore()
    for nbr in (left, right, diag):
        pl.semaphore_signal(barrier, inc=1, device_id=(nbr,),
                            device_id_type=pl.DeviceIdType.MESH)
    pl.semaphore_wait(barrier, 3)

    def ag_rdma(idx, src, dst, tgt):
        return pltpu.make_async_remote_copy(
            src_ref=src, dst_ref=dst,
            send_sem=ag_send.at[idx], recv_sem=ag_recv.at[idx],
            device_id=(tgt,), device_id_type=pl.DeviceIdType.MESH)

    def wo_rdma(idx, src, dst_slot, tgt):
        return pltpu.make_async_remote_copy(
            src_ref=src, dst_ref=wo_all_ref.at[dst_slot],
            send_sem=wo_send.at[idx], recv_sem=wo_recv.at[idx],
            device_id=(tgt,), device_id_type=pl.DeviceIdType.MESH)

    x16_ref[...] = x_ref[...].astype(BF16)
    cw1 = ag_rdma(CW1, x16_ref, xg_ref.at[1], right)
    ccw1 = ag_rdma(CCW1, x16_ref, xg_ref.at[3], left)
    cw1.start()
    ccw1.start()
    wo16_ref[...] = wo_ref[...].astype(BF16)
    w_cw1 = wo_rdma(CW1, wo16_ref, 1, right)
    w_ccw1 = wo_rdma(CCW1, wo16_ref, 3, left)
    w_cw1.start()
    w_ccw1.start()

    wqkv_ref[:, pl.ds(0, HD_LOC)] = wq_ref[...].astype(BF16)
    wqkv_ref[:, pl.ds(HD_LOC, HD_LOC)] = wk_ref[...].astype(BF16)
    wqkv_ref[:, pl.ds(2 * HD_LOC, HD_LOC)] = wv_ref[...].astype(BF16)
    rowp = lax.broadcasted_iota(jnp.int32, (SQ, HD_LOC), 0).astype(F32)
    lane = lax.broadcasted_iota(jnp.int32, (SQ, HD_LOC), 1)
    k2 = ((lane % DH) // 2).astype(F32)
    theta = rowp * jnp.exp(k2 * (-2.0 / DH) * jnp.log(10000.0))
    cos_ref[...] = jnp.cos(theta)
    sin_ref[...] = jnp.sin(theta)
    ar = lax.broadcasted_iota(jnp.int32, (HD_LOC, HD_LOC), 0)
    ac = lax.broadcasted_iota(jnp.int32, (HD_LOC, HD_LOC), 1)
    rmat = jnp.where(
        (ar == ac + 1) & (ar % 2 == 1), -1.0,
        jnp.where((ac == ar + 1) & (ar % 2 == 0), 1.0, 0.0),
    ).astype(F32)

    def proj(r):
        blk = lax.rem(my + (N_DEV - r), N_DEV)
        row0 = blk * SQ_SHARD
        cos_b = cos_ref[pl.ds(row0, SQ_SHARD), :]
        sin_b = sin_ref[pl.ds(row0, SQ_SHARD), :]
        for b in range(B):
            xb = x16_ref[b] if r == 0 else xg_ref[r, b]
            qkv = jnp.dot(xb, wqkv_ref[...], preferred_element_type=F32)
            q = qkv[:, 0:HD_LOC]
            k = qkv[:, HD_LOC:2 * HD_LOC]
            qp_ref[b, pl.ds(row0, SQ_SHARD), :] = (
                q * cos_b + jnp.dot(q, rmat, preferred_element_type=F32)
                * sin_b).astype(BF16)
            kp_ref[b, pl.ds(row0, SQ_SHARD), :] = (
                k * cos_b + jnp.dot(k, rmat, preferred_element_type=F32)
                * sin_b).astype(BF16)
            vp_ref[b, pl.ds(row0, SQ_SHARD), :] = (
                qkv[:, 2 * HD_LOC:3 * HD_LOC]).astype(BF16)

    proj(0)
    cw1.wait_recv()
    cw2 = ag_rdma(CW2, xg_ref.at[1, 0], xg_ref.at[2, 0], right)
    cw2.start()
    ccw1.wait_recv()
    ccw2 = ag_rdma(CCW2, xg_ref.at[3, 1], xg_ref.at[2, 1], left)
    ccw2.start()
    proj(1)
    proj(3)
    w_cw1.wait_recv()
    w_cw2 = wo_rdma(CW2, wo_all_ref.at[1], 2, right)
    w_cw2.start()
    cw2.wait_recv()
    ccw2.wait_recv()
    proj(2)

    def attn_block(r, b):
        blk = lax.rem(my + (N_DEV - r), N_DEV)
        row0 = blk * SQ_SHARD
        for h in range(H_LOC):
            q = qp_ref[b, pl.ds(row0, SQ_SHARD), pl.ds(h * DH, DH)]
            k = kp_ref[b, :, pl.ds(h * DH, DH)]
            v = vp_ref[b, :, pl.ds(h * DH, DH)]
            s = lax.dot_general(
                q, k, (((1,), (1,)), ((), ())),
                preferred_element_type=F32) * 0.125
            w = jnp.exp(s)
            w = (w / jnp.sum(w, axis=1, keepdims=True)).astype(BF16)
            ctxa_ref[r, b, :, pl.ds(h * DH, DH)] = jnp.dot(
                w, v, preferred_element_type=F32).astype(BF16)

    def a2a_rdma(idx, src_slot, rx_slot, tgt):
        return pltpu.make_async_remote_copy(
            src_ref=ctxa_ref.at[src_slot], dst_ref=rx_ref.at[rx_slot],
            send_sem=a2a_send.at[idx], recv_sem=a2a_recv.at[idx],
            device_id=(tgt,), device_id_type=pl.DeviceIdType.MESH)

    attn_block(2, 0)
    attn_block(2, 1)
    s_diag = a2a_rdma(FROM_DIAG, 2, FROM_DIAG, diag)
    s_diag.start()
    attn_block(1, 0)
    attn_block(1, 1)
    s_left = a2a_rdma(FROM_RIGHT, 1, FROM_RIGHT, left)
    s_left.start()
    attn_block(3, 0)
    attn_block(3, 1)
    s_right = a2a_rdma(FROM_LEFT, 3, FROM_LEFT, right)
    s_right.start()
    attn_block(0, 0)
    attn_block(0, 1)

    w_ccw1.wait_recv()
    w_cw2.wait_recv()
    for b in range(B):
        out_ref[b] = jnp.dot(ctxa_ref[0, b], wo16_ref[...],
                             preferred_element_type=F32)

    for rx_slot, wo_slot in ((FROM_LEFT, 1), (FROM_DIAG, 2),
                             (FROM_RIGHT, 3)):
        rcv = pltpu.make_async_remote_copy(
            src_ref=ctxa_ref.at[0], dst_ref=rx_ref.at[rx_slot],
            send_sem=a2a_send.at[rx_slot], recv_sem=a2a_recv.at[rx_slot],
            device_id=(my,), device_id_type=pl.DeviceIdType.MESH)
        rcv.wait_recv()
        for b in range(B):
            out_ref[b] = out_ref[b] + jnp.dot(
                rx_ref[rx_slot, b], wo_all_ref[wo_slot],
                preferred_element_type=F32)

    for d in (cw1, ccw1, cw2, ccw2, w_cw1, w_ccw1, w_cw2,
              s_diag, s_left, s_right):
        d.wait_send()


def kernel(x, Wq, Wk, Wv, Wo):
    return pl.pallas_call(
        _body,
        out_shape=jax.ShapeDtypeStruct((B, SQ_SHARD, D_MODEL), F32),
        in_specs=[pl.BlockSpec(memory_space=pltpu.VMEM)] * 5,
        out_specs=pl.BlockSpec(memory_space=pltpu.VMEM),
        scratch_shapes=[
            pltpu.VMEM((B, SQ_SHARD, D_MODEL), BF16),
            pltpu.VMEM((N_DEV, B, SQ_SHARD, D_MODEL), BF16),
            pltpu.VMEM((D_MODEL, 3 * HD_LOC), BF16),
            pltpu.VMEM((HD_LOC, D_MODEL), BF16),
            pltpu.VMEM((N_DEV, HD_LOC, D_MODEL), BF16),
            pltpu.VMEM((B, SQ, HD_LOC), BF16),
            pltpu.VMEM((B, SQ, HD_LOC), BF16),
            pltpu.VMEM((B, SQ, HD_LOC), BF16),
            pltpu.VMEM((N_DEV, B, SQ_SHARD, HD_LOC), BF16),
            pltpu.VMEM((3, B, SQ_SHARD, HD_LOC), BF16),
            pltpu.VMEM((SQ, HD_LOC), F32),
            pltpu.VMEM((SQ, HD_LOC), F32),
            pltpu.SemaphoreType.DMA((N_DEV,)),
            pltpu.SemaphoreType.DMA((N_DEV,)),
            pltpu.SemaphoreType.DMA((3,)),
            pltpu.SemaphoreType.DMA((3,)),
            pltpu.SemaphoreType.DMA((3,)),
            pltpu.SemaphoreType.DMA((3,)),
        ],
        compiler_params=pltpu.CompilerParams(
            collective_id=0, vmem_limit_bytes=100 * 1024 * 1024),
    )(x, Wq, Wk, Wv, Wo)
